# Initial kernel scaffold; baseline (speedup 1.0000x reference)
import jax
import jax.numpy as jnp
from jax.experimental import pallas as pl


def kernel(docs, table, W, b):
    raise NotImplementedError("write your pallas kernel here")


# speedup vs baseline: 7.1267x; 7.1267x over previous
import functools

import jax
import jax.numpy as jnp
from jax import lax
from jax.experimental import pallas as pl
from jax.experimental.pallas import tpu as pltpu
from jax.experimental.pallas import tpu_sc as plsc

L = 200
B = 4096
EMB = 64
NCLS = 50

NC = 2
NS = 16
NW = NC * NS
BPW = B // NW


def _sc_pool_body(docs_hbm, table_hbm, out_hbm, slab_v, rows_v, acc_v, sem):
    wid = lax.axis_index("s") * NC + lax.axis_index("c")
    base = wid * BPW

    pltpu.sync_copy(docs_hbm.at[:, pl.ds(base, BPW)], slab_v)

    zeros = jnp.zeros((16,), jnp.float32)

    def zero_body(bb, c2):
        for e in range(EMB // 16):
            acc_v[bb, pl.ds(e * 16, 16)] = zeros
        return c2

    lax.fori_loop(0, BPW, zero_body, 0, unroll=8)

    def step(l, carry):
        pltpu.async_copy(table_hbm.at[slab_v.at[l]], rows_v, sem).wait()

        def bb_body(bb, c2):
            for e in range(EMB // 16):
                sl = pl.ds(e * 16, 16)
                plsc.addupdate(acc_v.at[bb, sl], rows_v[bb, sl])
            return c2

        lax.fori_loop(0, BPW, bb_body, 0, unroll=8)
        return carry

    lax.fori_loop(0, L, step, 0)

    pltpu.sync_copy(acc_v, out_hbm.at[pl.ds(base, BPW)])


@functools.partial(jax.jit, static_argnames=())
def _sc_pool(docs, table):
    mesh = plsc.VectorSubcoreMesh(core_axis_name="c", subcore_axis_name="s")
    f = pl.kernel(
        _sc_pool_body,
        out_type=jax.ShapeDtypeStruct((B, EMB), jnp.float32),
        mesh=mesh,
        scratch_types=[
            pltpu.VMEM((L, BPW), jnp.int32),
            pltpu.VMEM((BPW, EMB), jnp.float32),
            pltpu.VMEM((BPW, EMB), jnp.float32),
            pltpu.SemaphoreType.DMA,
        ],
        compiler_params=pltpu.CompilerParams(use_tc_tiling_on_sc=False),
    )
    return f(docs, table)


def _tc_head_body(s_ref, w_ref, b_ref, o_ref):
    cbow = s_ref[...] * (1.0 / L)
    o_ref[...] = (
        lax.dot_general(
            cbow, w_ref[...],
            (((1,), (1,)), ((), ())),
            preferred_element_type=jnp.float32,
        )
        + b_ref[...]
    )


def _tc_head(sums, W, b):
    return pl.pallas_call(
        _tc_head_body,
        out_shape=jax.ShapeDtypeStruct((B, NCLS), jnp.float32),
    )(sums, W, b.reshape(1, NCLS))


def kernel(docs, table, W, b):
    sums = _sc_pool(docs, table)
    return _tc_head(sums, W, b)


# speedup vs baseline: 10.0059x; 1.4040x over previous
import functools

import jax
import jax.numpy as jnp
from jax import lax
from jax.experimental import pallas as pl
from jax.experimental.pallas import tpu as pltpu
from jax.experimental.pallas import tpu_sc as plsc

L = 200
B = 4096
EMB = 64
NCLS = 50

NC = 2
NS = 16
NW = NC * NS
BPW = B // NW


def _sc_pool_body(docs_hbm, table_hbm, out_hbm, slab_v, rows_v, acc_v, sem):
    wid = lax.axis_index("s") * NC + lax.axis_index("c")
    base = wid * BPW

    pltpu.sync_copy(docs_hbm.at[:, pl.ds(base, BPW)], slab_v)

    zeros = jnp.zeros((16,), jnp.float32)

    def zero_body(bb, c2):
        for e in range(EMB // 16):
            acc_v[bb, pl.ds(e * 16, 16)] = zeros
        return c2

    lax.fori_loop(0, BPW, zero_body, 0, unroll=8)

    def step(l, carry):
        pltpu.async_copy(table_hbm.at[slab_v.at[l]], acc_v, sem, add=True).wait()
        return carry

    lax.fori_loop(0, L, step, 0)

    pltpu.sync_copy(acc_v, out_hbm.at[pl.ds(base, BPW)])


@functools.partial(jax.jit, static_argnames=())
def _sc_pool(docs, table):
    mesh = plsc.VectorSubcoreMesh(core_axis_name="c", subcore_axis_name="s")
    f = pl.kernel(
        _sc_pool_body,
        out_type=jax.ShapeDtypeStruct((B, EMB), jnp.float32),
        mesh=mesh,
        scratch_types=[
            pltpu.VMEM((L, BPW), jnp.int32),
            pltpu.VMEM((BPW, EMB), jnp.float32),
            pltpu.VMEM((BPW, EMB), jnp.float32),
            pltpu.SemaphoreType.DMA,
        ],
        compiler_params=pltpu.CompilerParams(use_tc_tiling_on_sc=False),
    )
    return f(docs, table)


def _tc_head_body(s_ref, w_ref, b_ref, o_ref):
    cbow = s_ref[...] * (1.0 / L)
    o_ref[...] = (
        lax.dot_general(
            cbow, w_ref[...],
            (((1,), (1,)), ((), ())),
            preferred_element_type=jnp.float32,
        )
        + b_ref[...]
    )


def _tc_head(sums, W, b):
    return pl.pallas_call(
        _tc_head_body,
        out_shape=jax.ShapeDtypeStruct((B, NCLS), jnp.float32),
    )(sums, W, b.reshape(1, NCLS))


def kernel(docs, table, W, b):
    sums = _sc_pool(docs, table)
    return _tc_head(sums, W, b)


# speedup vs baseline: 17.1170x; 1.7107x over previous
import functools

import jax
import jax.numpy as jnp
from jax import lax
from jax.experimental import pallas as pl
from jax.experimental.pallas import tpu as pltpu
from jax.experimental.pallas import tpu_sc as plsc

L = 200
B = 4096
EMB = 64
NCLS = 50

NC = 2
NS = 16
NW = NC * NS
BPW = B // NW


K = 20
NG = L // K


def _sc_pool_body(docs_hbm, table_hbm, out_hbm, slab_v, acc_v, sem):
    wid = lax.axis_index("s") * NC + lax.axis_index("c")
    base = wid * BPW

    pltpu.sync_copy(docs_hbm.at[:, pl.ds(base, BPW)], slab_v)

    zeros = jnp.zeros((16,), jnp.float32)

    def zero_body(bb, c2):
        for e in range(EMB // 16):
            acc_v[bb, pl.ds(e * 16, 16)] = zeros
        return c2

    lax.fori_loop(0, BPW, zero_body, 0, unroll=8)

    def group(g, carry):
        l0 = g * K
        cps = [
            pltpu.async_copy(
                table_hbm.at[slab_v.at[l0 + j]], acc_v, sem, add=True
            )
            for j in range(K)
        ]
        for cp in cps:
            cp.wait()
        return carry

    lax.fori_loop(0, NG, group, 0)

    pltpu.sync_copy(acc_v, out_hbm.at[pl.ds(base, BPW)])


@functools.partial(jax.jit, static_argnames=())
def _sc_pool(docs, table):
    mesh = plsc.VectorSubcoreMesh(core_axis_name="c", subcore_axis_name="s")
    f = pl.kernel(
        _sc_pool_body,
        out_type=jax.ShapeDtypeStruct((B, EMB), jnp.float32),
        mesh=mesh,
        scratch_types=[
            pltpu.VMEM((L, BPW), jnp.int32),
            pltpu.VMEM((BPW, EMB), jnp.float32),
            pltpu.SemaphoreType.DMA,
        ],
        compiler_params=pltpu.CompilerParams(use_tc_tiling_on_sc=False),
    )
    return f(docs, table)


def _tc_head_body(s_ref, w_ref, b_ref, o_ref):
    cbow = s_ref[...] * (1.0 / L)
    o_ref[...] = (
        lax.dot_general(
            cbow, w_ref[...],
            (((1,), (1,)), ((), ())),
            preferred_element_type=jnp.float32,
        )
        + b_ref[...]
    )


def _tc_head(sums, W, b):
    return pl.pallas_call(
        _tc_head_body,
        out_shape=jax.ShapeDtypeStruct((B, NCLS), jnp.float32),
    )(sums, W, b.reshape(1, NCLS))


def kernel(docs, table, W, b):
    sums = _sc_pool(docs, table)
    return _tc_head(sums, W, b)


# speedup vs baseline: 17.1197x; 1.0002x over previous
import functools

import jax
import jax.numpy as jnp
from jax import lax
from jax.experimental import pallas as pl
from jax.experimental.pallas import tpu as pltpu
from jax.experimental.pallas import tpu_sc as plsc

L = 200
B = 4096
EMB = 64
NCLS = 50

NC = 2
NS = 16
NW = NC * NS
BPW = B // NW


K = 24


def _sc_pool_body(docs_hbm, table_hbm, out_hbm, slab_v, acc_v, sem):
    wid = lax.axis_index("s") * NC + lax.axis_index("c")
    base = wid * BPW

    pltpu.sync_copy(docs_hbm.at[:, pl.ds(base, BPW)], slab_v)

    zeros = jnp.zeros((16,), jnp.float32)

    def zero_body(bb, c2):
        for e in range(EMB // 16):
            acc_v[bb, pl.ds(e * 16, 16)] = zeros
        return c2

    lax.fori_loop(0, BPW, zero_body, 0, unroll=8)

    cps = []
    for l in range(L):
        cps.append(
            pltpu.async_copy(
                table_hbm.at[slab_v.at[l]], acc_v, sem, add=True
            )
        )
        if l >= K:
            cps[l - K].wait()
    for l in range(L - K, L):
        cps[l].wait()

    pltpu.sync_copy(acc_v, out_hbm.at[pl.ds(base, BPW)])


@functools.partial(jax.jit, static_argnames=())
def _sc_pool(docs, table):
    mesh = plsc.VectorSubcoreMesh(core_axis_name="c", subcore_axis_name="s")
    f = pl.kernel(
        _sc_pool_body,
        out_type=jax.ShapeDtypeStruct((B, EMB), jnp.float32),
        mesh=mesh,
        scratch_types=[
            pltpu.VMEM((L, BPW), jnp.int32),
            pltpu.VMEM((BPW, EMB), jnp.float32),
            pltpu.SemaphoreType.DMA,
        ],
        compiler_params=pltpu.CompilerParams(use_tc_tiling_on_sc=False),
    )
    return f(docs, table)


def _tc_head_body(s_ref, w_ref, b_ref, o_ref):
    cbow = s_ref[...] * (1.0 / L)
    o_ref[...] = (
        lax.dot_general(
            cbow, w_ref[...],
            (((1,), (1,)), ((), ())),
            preferred_element_type=jnp.float32,
        )
        + b_ref[...]
    )


def _tc_head(sums, W, b):
    return pl.pallas_call(
        _tc_head_body,
        out_shape=jax.ShapeDtypeStruct((B, NCLS), jnp.float32),
    )(sums, W, b.reshape(1, NCLS))


def kernel(docs, table, W, b):
    sums = _sc_pool(docs, table)
    return _tc_head(sums, W, b)
